# Initial kernel scaffold; baseline (speedup 1.0000x reference)
import jax
import jax.numpy as jnp
from jax.experimental import pallas as pl


def kernel(hidden_states, position_ids, ln1_w, ln2_w, wq, bq, wk, bk, wv, bv, wo, bo, sinks, router_kernel, router_bias, gate_up_proj, gate_up_proj_bias, down_proj, down_proj_bias):
    raise NotImplementedError("write your pallas kernel here")


# speedup vs baseline: 6.3634x; 6.3634x over previous
import functools

import jax
import jax.numpy as jnp
import numpy as np
from jax.experimental import pallas as pl
from jax.experimental.pallas import tpu as pltpu

B, S, HID = 16, 4, 768
NH, HD = 12, 64
E, TOPK, FF = 8, 2, 768
T = B * S
EPS = 1e-05
THETA = 10000.0
ALPHA = 1.702
LIMIT = 7.0


def _build_consts():
    i = np.arange(NH * HD)[:, None]
    k = np.arange(NH * HD)[None, :]
    ji = i % HD
    jk = k % HD
    same_head = (i // HD) == (k // HD)
    P = np.where(same_head & (jk < HD // 2) & (ji == jk + HD // 2), -1.0, 0.0)
    P = P + np.where(same_head & (jk >= HD // 2) & (ji == jk - HD // 2), 1.0, 0.0)
    r = np.arange(2 * FF)[:, None]
    c = np.arange(FF)[None, :]
    se = (r == 2 * c).astype(np.float32)
    so = (r == 2 * c + 1).astype(np.float32)
    return (jnp.asarray(P, jnp.float32), jnp.asarray(se), jnp.asarray(so))


def _decoder_kernel(x_ref, cos_ref, sin_ref, ln1_ref, ln2_ref, wq_ref, bq_ref,
                    wk_ref, bk_ref, wv_ref, bv_ref, wo_ref, bo_ref, sinks_ref,
                    rkt_ref, rb_ref, wgu_ref, bg_ref, bu_ref, wd_ref, bd_ref,
                    p_ref, se_ref, so_ref, out_ref,
                    hid_s, h2_s, attn_s, route_s):
    e = pl.program_id(0)

    @pl.when(e == 0)
    def _attention_and_router():
        x = x_ref[...]
        v1 = jnp.mean(jnp.square(x), axis=-1, keepdims=True)
        h = x * jax.lax.rsqrt(v1 + EPS) * ln1_ref[...]
        q = jnp.dot(h, wq_ref[...], preferred_element_type=jnp.float32) + bq_ref[...]
        k = jnp.dot(h, wk_ref[...], preferred_element_type=jnp.float32) + bk_ref[...]
        v = jnp.dot(h, wv_ref[...], preferred_element_type=jnp.float32) + bv_ref[...]
        cos = cos_ref[...]
        sin = sin_ref[...]
        p_mat = p_ref[...]
        qr = q * cos + jnp.dot(q, p_mat, preferred_element_type=jnp.float32) * sin
        kr = k * cos + jnp.dot(k, p_mat, preferred_element_type=jnp.float32) * sin

        row = jax.lax.broadcasted_iota(jnp.int32, (T, T), 0)
        col = jax.lax.broadcasted_iota(jnp.int32, (T, T), 1)
        allowed = (row // S == col // S) & (col <= row)
        scale = 1.0 / np.sqrt(HD)
        for hh in range(NH):
            sl = slice(HD * hh, HD * (hh + 1))
            g = jax.lax.dot_general(qr[:, sl], kr[:, sl],
                                    (((1,), (1,)), ((), ())),
                                    preferred_element_type=jnp.float32) * scale
            g = jnp.where(allowed, g, -1e30)
            m = jnp.max(g, axis=1, keepdims=True)
            sk = sinks_ref[0:1, hh:hh + 1]
            m2 = jnp.maximum(m, sk)
            pexp = jnp.exp(g - m2)
            denom = jnp.sum(pexp, axis=1, keepdims=True) + jnp.exp(sk - m2)
            probs = pexp / denom
            attn_s[:, sl] = jnp.dot(probs, v[:, sl],
                                    preferred_element_type=jnp.float32)

        ao = jnp.dot(attn_s[...], wo_ref[...],
                     preferred_element_type=jnp.float32) + bo_ref[...]
        hid = x + ao
        hid_s[...] = hid
        v2 = jnp.mean(jnp.square(hid), axis=-1, keepdims=True)
        h2 = hid * jax.lax.rsqrt(v2 + EPS) * ln2_ref[...]
        h2_s[...] = h2

        logits = jnp.dot(h2, rkt_ref[...],
                         preferred_element_type=jnp.float32) + rb_ref[...]
        idx = jax.lax.broadcasted_iota(jnp.int32, (T, E), 1).astype(jnp.float32)
        m1 = jnp.max(logits, axis=1, keepdims=True)
        i1 = jnp.min(jnp.where(logits == m1, idx, 1e9), axis=1, keepdims=True)
        l2 = jnp.where(idx == i1, -1e30, logits)
        m2r = jnp.max(l2, axis=1, keepdims=True)
        i2 = jnp.min(jnp.where(l2 == m2r, idx, 1e9), axis=1, keepdims=True)
        em2 = jnp.exp(m2r - m1)
        p1 = 1.0 / (1.0 + em2)
        p2 = em2 / (1.0 + em2)
        route_s[:, 0:1] = i1
        route_s[:, 1:2] = p1
        route_s[:, 2:3] = i2
        route_s[:, 3:4] = p2

    h2 = h2_s[...]
    g2 = jnp.dot(h2, wgu_ref[0], preferred_element_type=jnp.float32)
    gate = jnp.dot(g2, se_ref[...], preferred_element_type=jnp.float32) + bg_ref[0]
    up = jnp.dot(g2, so_ref[...], preferred_element_type=jnp.float32) + bu_ref[0]
    gate = jnp.minimum(gate, LIMIT)
    up = jnp.clip(up, -LIMIT, LIMIT)
    glu = gate * jax.nn.sigmoid(gate * ALPHA)
    fused = (up + 1.0) * glu
    nxt = jnp.dot(fused, wd_ref[0], preferred_element_type=jnp.float32) + bd_ref[0]
    ef = e.astype(jnp.float32)
    w_e = (jnp.where(route_s[:, 0:1] == ef, route_s[:, 1:2], 0.0)
           + jnp.where(route_s[:, 2:3] == ef, route_s[:, 3:4], 0.0))
    contrib = nxt * w_e

    @pl.when(e == 0)
    def _init():
        out_ref[...] = hid_s[...] + contrib

    @pl.when(e != 0)
    def _acc():
        out_ref[...] += contrib


@functools.partial(jax.jit, static_argnames=("interpret",))
def kernel(hidden_states, position_ids, ln1_w, ln2_w, wq, bq, wk, bk, wv, bv,
           wo, bo, sinks, router_kernel, router_bias, gate_up_proj,
           gate_up_proj_bias, down_proj, down_proj_bias, interpret=False):
    x = hidden_states.reshape(T, HID)
    posf = position_ids.astype(jnp.float32).reshape(T, 1)
    jm = jnp.asarray(np.arange(NH * HD) % (HD // 2), jnp.float32)[None, :]
    inv = jnp.exp(jm * (-2.0 / HD) * np.log(THETA))
    ang = posf * inv
    cosf = jnp.cos(ang)
    sinf = jnp.sin(ang)
    p_mat, sel_e, sel_o = _build_consts()
    bgate = gate_up_proj_bias[:, 0::2]
    bup = gate_up_proj_bias[:, 1::2]

    const = lambda r: pl.BlockSpec(r, lambda e: (0,) * len(r))
    per_e3 = lambda a, b: pl.BlockSpec((1, a, b), lambda e: (e, 0, 0))
    per_e2 = lambda a: pl.BlockSpec((1, 1, a), lambda e: (e, 0, 0))

    out = pl.pallas_call(
        _decoder_kernel,
        grid=(E,),
        in_specs=[
            const((T, HID)),
            const((T, HID)),
            const((T, HID)),
            const((1, HID)),
            const((1, HID)),
            const((HID, NH * HD)),
            const((1, NH * HD)),
            const((HID, NH * HD)),
            const((1, NH * HD)),
            const((HID, NH * HD)),
            const((1, NH * HD)),
            const((NH * HD, HID)),
            const((1, HID)),
            const((1, NH)),
            const((HID, E)),
            const((1, E)),
            per_e3(HID, 2 * FF),
            per_e2(FF),
            per_e2(FF),
            per_e3(FF, HID),
            per_e2(HID),
            const((NH * HD, NH * HD)),
            const((2 * FF, FF)),
            const((2 * FF, FF)),
        ],
        out_specs=pl.BlockSpec((T, HID), lambda e: (0, 0)),
        out_shape=jax.ShapeDtypeStruct((T, HID), jnp.float32),
        scratch_shapes=[
            pltpu.VMEM((T, HID), jnp.float32),
            pltpu.VMEM((T, HID), jnp.float32),
            pltpu.VMEM((T, NH * HD), jnp.float32),
            pltpu.VMEM((T, 8), jnp.float32),
        ],
        interpret=interpret,
    )(x, cosf, sinf, ln1_w[None, :], ln2_w[None, :], wq, bq[None, :], wk,
      bk[None, :], wv, bv[None, :], wo, bo[None, :], sinks[None, :],
      router_kernel.T, router_bias[None, :], gate_up_proj,
      bgate[:, None, :], bup[:, None, :], down_proj,
      down_proj_bias[:, None, :], p_mat, sel_e, sel_o)
    return out.reshape(B, S, HID)


# speedup vs baseline: 7.3001x; 1.1472x over previous
import functools

import jax
import jax.numpy as jnp
import numpy as np
from jax.experimental import pallas as pl
from jax.experimental.pallas import tpu as pltpu

B, S, HID = 16, 4, 768
NH, HD = 12, 64
E, TOPK, FF = 8, 2, 768
T = B * S
EPS = 1e-05
THETA = 10000.0
ALPHA = 1.702
LIMIT = 7.0


def _decoder_kernel(x_ref, cos_ref, sin_ref, ln1_ref, ln2_ref, wq_ref, bq_ref,
                    wk_ref, bk_ref, wv_ref, bv_ref, wo_ref, bo_ref, sinks_ref,
                    rkt_ref, rb_ref, wgu_ref, bgu_ref, wd_ref, bd_ref, out_ref,
                    hid_s, h2_s, attn_s, route_s, se_s):
    e = pl.program_id(0)

    @pl.when(e == 0)
    def _attention_and_router():
        rr = jax.lax.broadcasted_iota(jnp.int32, (2 * FF, FF), 0)
        cc = jax.lax.broadcasted_iota(jnp.int32, (2 * FF, FF), 1)
        se_s[...] = jnp.where(rr == 2 * cc, 1.0, 0.0)

        pi = jax.lax.broadcasted_iota(jnp.int32, (NH * HD, NH * HD), 0)
        pk = jax.lax.broadcasted_iota(jnp.int32, (NH * HD, NH * HD), 1)
        ji = pi % HD
        jk = pk % HD
        same_head = (pi // HD) == (pk // HD)
        p_mat = jnp.where(same_head & (jk < HD // 2) & (ji == jk + HD // 2),
                          -1.0, 0.0)
        p_mat = p_mat + jnp.where(
            same_head & (jk >= HD // 2) & (ji == jk - HD // 2), 1.0, 0.0)
        x = x_ref[...]
        v1 = jnp.mean(jnp.square(x), axis=-1, keepdims=True)
        h = x * jax.lax.rsqrt(v1 + EPS) * ln1_ref[...]
        q = jnp.dot(h, wq_ref[...], preferred_element_type=jnp.float32) + bq_ref[...]
        k = jnp.dot(h, wk_ref[...], preferred_element_type=jnp.float32) + bk_ref[...]
        v = jnp.dot(h, wv_ref[...], preferred_element_type=jnp.float32) + bv_ref[...]
        cos = cos_ref[...]
        sin = sin_ref[...]
        qr = q * cos + jnp.dot(q, p_mat, preferred_element_type=jnp.float32) * sin
        kr = k * cos + jnp.dot(k, p_mat, preferred_element_type=jnp.float32) * sin

        row = jax.lax.broadcasted_iota(jnp.int32, (T, T), 0)
        col = jax.lax.broadcasted_iota(jnp.int32, (T, T), 1)
        allowed = (row // S == col // S) & (col <= row)
        scale = 1.0 / np.sqrt(HD)
        for hh in range(NH):
            sl = slice(HD * hh, HD * (hh + 1))
            g = jax.lax.dot_general(qr[:, sl], kr[:, sl],
                                    (((1,), (1,)), ((), ())),
                                    preferred_element_type=jnp.float32) * scale
            g = jnp.where(allowed, g, -1e30)
            m = jnp.max(g, axis=1, keepdims=True)
            sk = sinks_ref[0:1, hh:hh + 1]
            m2 = jnp.maximum(m, sk)
            pexp = jnp.exp(g - m2)
            denom = jnp.sum(pexp, axis=1, keepdims=True) + jnp.exp(sk - m2)
            probs = pexp / denom
            attn_s[:, sl] = jnp.dot(probs, v[:, sl],
                                    preferred_element_type=jnp.float32)

        ao = jnp.dot(attn_s[...], wo_ref[...],
                     preferred_element_type=jnp.float32) + bo_ref[...]
        hid = x + ao
        hid_s[...] = hid
        v2 = jnp.mean(jnp.square(hid), axis=-1, keepdims=True)
        h2 = hid * jax.lax.rsqrt(v2 + EPS) * ln2_ref[...]
        h2_s[...] = h2

        logits = jnp.dot(h2, rkt_ref[...],
                         preferred_element_type=jnp.float32) + rb_ref[...]
        idx = jax.lax.broadcasted_iota(jnp.int32, (T, E), 1).astype(jnp.float32)
        m1 = jnp.max(logits, axis=1, keepdims=True)
        i1 = jnp.min(jnp.where(logits == m1, idx, 1e9), axis=1, keepdims=True)
        l2 = jnp.where(idx == i1, -1e30, logits)
        m2r = jnp.max(l2, axis=1, keepdims=True)
        i2 = jnp.min(jnp.where(l2 == m2r, idx, 1e9), axis=1, keepdims=True)
        em2 = jnp.exp(m2r - m1)
        p1 = 1.0 / (1.0 + em2)
        p2 = em2 / (1.0 + em2)
        route_s[:, 0:1] = i1
        route_s[:, 1:2] = p1
        route_s[:, 2:3] = i2
        route_s[:, 3:4] = p2

    h2 = h2_s[...]
    g2 = jnp.dot(h2, wgu_ref[0],
                 preferred_element_type=jnp.float32) + bgu_ref[0]
    gl = jnp.minimum(g2, LIMIT)
    gl = gl * jax.nn.sigmoid(gl * ALPHA)
    uc = jnp.clip(g2, -LIMIT, LIMIT) + 1.0
    uc_shift = jnp.concatenate([uc[:, 1:], uc[:, :1]], axis=1)
    fused = jnp.dot(gl * uc_shift, se_s[...],
                    preferred_element_type=jnp.float32)
    nxt = jnp.dot(fused, wd_ref[0], preferred_element_type=jnp.float32) + bd_ref[0]
    ef = e.astype(jnp.float32)
    w_e = (jnp.where(route_s[:, 0:1] == ef, route_s[:, 1:2], 0.0)
           + jnp.where(route_s[:, 2:3] == ef, route_s[:, 3:4], 0.0))
    contrib = nxt * w_e

    @pl.when(e == 0)
    def _init():
        out_ref[...] = hid_s[...] + contrib

    @pl.when(e != 0)
    def _acc():
        out_ref[...] += contrib


@functools.partial(jax.jit, static_argnames=("interpret",))
def kernel(hidden_states, position_ids, ln1_w, ln2_w, wq, bq, wk, bk, wv, bv,
           wo, bo, sinks, router_kernel, router_bias, gate_up_proj,
           gate_up_proj_bias, down_proj, down_proj_bias, interpret=False):
    x = hidden_states.reshape(T, HID)
    posf = position_ids.astype(jnp.float32).reshape(T, 1)
    jm = jnp.asarray(np.arange(NH * HD) % (HD // 2), jnp.float32)[None, :]
    inv = jnp.exp(jm * (-2.0 / HD) * np.log(THETA))
    ang = posf * inv
    cosf = jnp.cos(ang)
    sinf = jnp.sin(ang)

    const = lambda r: pl.BlockSpec(r, lambda e: (0,) * len(r))
    per_e3 = lambda a, b: pl.BlockSpec((1, a, b), lambda e: (e, 0, 0))
    per_e2 = lambda a: pl.BlockSpec((1, 1, a), lambda e: (e, 0, 0))

    out = pl.pallas_call(
        _decoder_kernel,
        grid=(E,),
        in_specs=[
            const((T, HID)),
            const((T, HID)),
            const((T, HID)),
            const((1, HID)),
            const((1, HID)),
            const((HID, NH * HD)),
            const((1, NH * HD)),
            const((HID, NH * HD)),
            const((1, NH * HD)),
            const((HID, NH * HD)),
            const((1, NH * HD)),
            const((NH * HD, HID)),
            const((1, HID)),
            const((1, NH)),
            const((HID, E)),
            const((1, E)),
            per_e3(HID, 2 * FF),
            per_e2(2 * FF),
            per_e3(FF, HID),
            per_e2(HID),
        ],
        out_specs=pl.BlockSpec((T, HID), lambda e: (0, 0)),
        out_shape=jax.ShapeDtypeStruct((T, HID), jnp.float32),
        scratch_shapes=[
            pltpu.VMEM((T, HID), jnp.float32),
            pltpu.VMEM((T, HID), jnp.float32),
            pltpu.VMEM((T, NH * HD), jnp.float32),
            pltpu.VMEM((T, 8), jnp.float32),
            pltpu.VMEM((2 * FF, FF), jnp.float32),
        ],
        interpret=interpret,
    )(x, cosf, sinf, ln1_w[None, :], ln2_w[None, :], wq, bq[None, :], wk,
      bk[None, :], wv, bv[None, :], wo, bo[None, :], sinks[None, :],
      router_kernel.T, router_bias[None, :], gate_up_proj,
      gate_up_proj_bias[:, None, :], down_proj,
      down_proj_bias[:, None, :])
    return out.reshape(B, S, HID)


# speedup vs baseline: 9.3248x; 1.2773x over previous
import functools

import jax
import jax.numpy as jnp
import numpy as np
from jax.experimental import pallas as pl
from jax.experimental.pallas import tpu as pltpu

B, S, HID = 16, 4, 768
NH, HD = 12, 64
E, TOPK, FF = 8, 2, 768
T = B * S
EPS = 1e-05
THETA = 10000.0
ALPHA = 1.702
LIMIT = 7.0
NSLOT = 4


def _decoder_kernel(x_ref, cos_ref, sin_ref, ln1_ref, ln2_ref, wq_ref, bq_ref,
                    wk_ref, bk_ref, wv_ref, bv_ref, wo_ref, bo_ref, sinks_ref,
                    rkt_ref, rb_ref, wgu_hbm, bgu_ref, wd_hbm, bd_ref, out_ref,
                    gup_buf, dwn_buf, se_s, gup_sem, dwn_sem):
    def start_expert(e):
        pltpu.make_async_copy(wgu_hbm.at[e], gup_buf.at[e % NSLOT],
                              gup_sem.at[e]).start()
        pltpu.make_async_copy(wd_hbm.at[e], dwn_buf.at[e % NSLOT],
                              dwn_sem.at[e]).start()

    def wait_expert(e):
        pltpu.make_async_copy(wgu_hbm.at[e], gup_buf.at[e % NSLOT],
                              gup_sem.at[e]).wait()
        pltpu.make_async_copy(wd_hbm.at[e], dwn_buf.at[e % NSLOT],
                              dwn_sem.at[e]).wait()

    for e in range(NSLOT - 1):
        start_expert(e)

    rr = jax.lax.broadcasted_iota(jnp.int32, (2 * FF, FF), 0)
    cc = jax.lax.broadcasted_iota(jnp.int32, (2 * FF, FF), 1)
    se_s[...] = jnp.where(rr == 2 * cc, 1.0, 0.0)

    pi = jax.lax.broadcasted_iota(jnp.int32, (NH * HD, NH * HD), 0)
    pk = jax.lax.broadcasted_iota(jnp.int32, (NH * HD, NH * HD), 1)
    ji = pi % HD
    jk = pk % HD
    same_head = (pi // HD) == (pk // HD)
    p_mat = jnp.where(same_head & (jk < HD // 2) & (ji == jk + HD // 2),
                      -1.0, 0.0)
    p_mat = p_mat + jnp.where(
        same_head & (jk >= HD // 2) & (ji == jk - HD // 2), 1.0, 0.0)

    x = x_ref[...]
    v1 = jnp.mean(jnp.square(x), axis=-1, keepdims=True)
    h = x * jax.lax.rsqrt(v1 + EPS) * ln1_ref[...]
    q = jnp.dot(h, wq_ref[...], preferred_element_type=jnp.float32) + bq_ref[...]
    k = jnp.dot(h, wk_ref[...], preferred_element_type=jnp.float32) + bk_ref[...]
    v = jnp.dot(h, wv_ref[...], preferred_element_type=jnp.float32) + bv_ref[...]
    cos = cos_ref[...]
    sin = sin_ref[...]
    qr = q * cos + jnp.dot(q, p_mat, preferred_element_type=jnp.float32) * sin
    kr = k * cos + jnp.dot(k, p_mat, preferred_element_type=jnp.float32) * sin

    row = jax.lax.broadcasted_iota(jnp.int32, (T, T), 0)
    col = jax.lax.broadcasted_iota(jnp.int32, (T, T), 1)
    allowed = (row // S == col // S) & (col <= row)
    scale = 1.0 / np.sqrt(HD)
    attn_cols = []
    for hh in range(NH):
        sl = slice(HD * hh, HD * (hh + 1))
        g = jax.lax.dot_general(qr[:, sl], kr[:, sl],
                                (((1,), (1,)), ((), ())),
                                preferred_element_type=jnp.float32) * scale
        g = jnp.where(allowed, g, -1e30)
        m = jnp.max(g, axis=1, keepdims=True)
        sk = sinks_ref[0:1, hh:hh + 1]
        m2 = jnp.maximum(m, sk)
        pexp = jnp.exp(g - m2)
        denom = jnp.sum(pexp, axis=1, keepdims=True) + jnp.exp(sk - m2)
        probs = pexp / denom
        attn_cols.append(jnp.dot(probs, v[:, sl],
                                 preferred_element_type=jnp.float32))
    attn = jnp.concatenate(attn_cols, axis=1)

    ao = jnp.dot(attn, wo_ref[...],
                 preferred_element_type=jnp.float32) + bo_ref[...]
    hid = x + ao
    v2 = jnp.mean(jnp.square(hid), axis=-1, keepdims=True)
    h2 = hid * jax.lax.rsqrt(v2 + EPS) * ln2_ref[...]

    logits = jnp.dot(h2, rkt_ref[...],
                     preferred_element_type=jnp.float32) + rb_ref[...]
    idx = jax.lax.broadcasted_iota(jnp.int32, (T, E), 1).astype(jnp.float32)
    m1 = jnp.max(logits, axis=1, keepdims=True)
    i1 = jnp.min(jnp.where(logits == m1, idx, 1e9), axis=1, keepdims=True)
    l2 = jnp.where(idx == i1, -1e30, logits)
    m2r = jnp.max(l2, axis=1, keepdims=True)
    i2 = jnp.min(jnp.where(l2 == m2r, idx, 1e9), axis=1, keepdims=True)
    em2 = jnp.exp(m2r - m1)
    p1 = 1.0 / (1.0 + em2)
    p2 = em2 / (1.0 + em2)

    acc = hid
    se_mat = se_s[...]
    for e in range(E):
        if e + NSLOT - 1 < E:
            start_expert(e + NSLOT - 1)
        wait_expert(e)
        g2 = jnp.dot(h2, gup_buf[e % NSLOT],
                     preferred_element_type=jnp.float32) + bgu_ref[e:e + 1, :]
        gl = jnp.minimum(g2, LIMIT)
        gl = gl * jax.nn.sigmoid(gl * ALPHA)
        uc = jnp.clip(g2, -LIMIT, LIMIT) + 1.0
        uc_shift = jnp.concatenate([uc[:, 1:], uc[:, :1]], axis=1)
        fused = jnp.dot(gl * uc_shift, se_mat,
                        preferred_element_type=jnp.float32)
        nxt = jnp.dot(fused, dwn_buf[e % NSLOT],
                      preferred_element_type=jnp.float32) + bd_ref[e:e + 1, :]
        ef = float(e)
        w_e = (jnp.where(i1 == ef, p1, 0.0) + jnp.where(i2 == ef, p2, 0.0))
        acc = acc + nxt * w_e
    out_ref[...] = acc


@functools.partial(jax.jit, static_argnames=("interpret",))
def kernel(hidden_states, position_ids, ln1_w, ln2_w, wq, bq, wk, bk, wv, bv,
           wo, bo, sinks, router_kernel, router_bias, gate_up_proj,
           gate_up_proj_bias, down_proj, down_proj_bias, interpret=False):
    x = hidden_states.reshape(T, HID)
    posf = position_ids.astype(jnp.float32).reshape(T, 1)
    jm = jnp.asarray(np.arange(NH * HD) % (HD // 2), jnp.float32)[None, :]
    inv = jnp.exp(jm * (-2.0 / HD) * np.log(THETA))
    ang = posf * inv
    cosf = jnp.cos(ang)
    sinf = jnp.sin(ang)

    vmem = pl.BlockSpec(memory_space=pltpu.VMEM)
    hbm = pl.BlockSpec(memory_space=pl.ANY)

    out = pl.pallas_call(
        _decoder_kernel,
        in_specs=[
            vmem,
            vmem,
            vmem,
            vmem,
            vmem,
            vmem,
            vmem,
            vmem,
            vmem,
            vmem,
            vmem,
            vmem,
            vmem,
            vmem,
            vmem,
            vmem,
            hbm,
            vmem,
            hbm,
            vmem,
        ],
        out_specs=pl.BlockSpec(memory_space=pltpu.VMEM),
        out_shape=jax.ShapeDtypeStruct((T, HID), jnp.float32),
        scratch_shapes=[
            pltpu.VMEM((NSLOT, HID, 2 * FF), jnp.float32),
            pltpu.VMEM((NSLOT, FF, HID), jnp.float32),
            pltpu.VMEM((2 * FF, FF), jnp.float32),
            pltpu.SemaphoreType.DMA((E,)),
            pltpu.SemaphoreType.DMA((E,)),
        ],
        interpret=interpret,
    )(x, cosf, sinf, ln1_w[None, :], ln2_w[None, :], wq, bq[None, :], wk,
      bk[None, :], wv, bv[None, :], wo, bo[None, :], sinks[None, :],
      router_kernel.T, router_bias[None, :], gate_up_proj,
      gate_up_proj_bias, down_proj, down_proj_bias)
    return out.reshape(B, S, HID)


# speedup vs baseline: 9.4034x; 1.0084x over previous
import jax
import jax.numpy as jnp
import numpy as np
from jax.experimental import pallas as pl
from jax.experimental.pallas import tpu as pltpu

B, S, HID = 16, 4, 768
NH, HD = 12, 64
E, TOPK, FF = 8, 2, 768
T = B * S
EPS = 1e-05
THETA = 10000.0
ALPHA = 1.702
LIMIT = 7.0
NSLOT = 4


def _decoder_kernel(x_ref, cos_ref, sin_ref, ln1_ref, ln2_ref, wq_ref, bq_ref,
                    wk_ref, bk_ref, wv_ref, bv_ref, wo_ref, bo_ref, sinks_ref,
                    rkt_ref, rb_ref, wgu_hbm, bgu_ref, wd_hbm, bd_ref, out_ref,
                    gup_buf, dwn_buf, se_s, gup_sem, dwn_sem):
    def start_expert(e):
        pltpu.make_async_copy(wgu_hbm.at[e], gup_buf.at[e % NSLOT],
                              gup_sem.at[e]).start()
        pltpu.make_async_copy(wd_hbm.at[e], dwn_buf.at[e % NSLOT],
                              dwn_sem.at[e]).start()

    def wait_expert(e):
        pltpu.make_async_copy(wgu_hbm.at[e], gup_buf.at[e % NSLOT],
                              gup_sem.at[e]).wait()
        pltpu.make_async_copy(wd_hbm.at[e], dwn_buf.at[e % NSLOT],
                              dwn_sem.at[e]).wait()

    for e in range(NSLOT - 1):
        start_expert(e)

    rr = jax.lax.broadcasted_iota(jnp.int32, (2 * FF, FF), 0)
    cc = jax.lax.broadcasted_iota(jnp.int32, (2 * FF, FF), 1)
    se_s[...] = jnp.where(rr == 2 * cc, 1.0, 0.0)

    pi = jax.lax.broadcasted_iota(jnp.int32, (NH * HD, NH * HD), 0)
    pk = jax.lax.broadcasted_iota(jnp.int32, (NH * HD, NH * HD), 1)
    ji = pi % HD
    jk = pk % HD
    same_head = (pi // HD) == (pk // HD)
    p_mat = jnp.where(same_head & (jk < HD // 2) & (ji == jk + HD // 2),
                      -1.0, 0.0)
    p_mat = p_mat + jnp.where(
        same_head & (jk >= HD // 2) & (ji == jk - HD // 2), 1.0, 0.0)

    x = x_ref[...]
    v1 = jnp.mean(jnp.square(x), axis=-1, keepdims=True)
    h = x * jax.lax.rsqrt(v1 + EPS) * ln1_ref[...]
    q = jnp.dot(h, wq_ref[...], preferred_element_type=jnp.float32) + bq_ref[...]
    k = jnp.dot(h, wk_ref[...], preferred_element_type=jnp.float32) + bk_ref[...]
    v = jnp.dot(h, wv_ref[...], preferred_element_type=jnp.float32) + bv_ref[...]
    cos = cos_ref[...]
    sin = sin_ref[...]
    qr = q * cos + jnp.dot(q, p_mat, preferred_element_type=jnp.float32) * sin
    kr = k * cos + jnp.dot(k, p_mat, preferred_element_type=jnp.float32) * sin

    row = jax.lax.broadcasted_iota(jnp.int32, (T, T), 0)
    col = jax.lax.broadcasted_iota(jnp.int32, (T, T), 1)
    allowed = (row // S == col // S) & (col <= row)
    scale = 1.0 / np.sqrt(HD)
    attn_cols = []
    for hh in range(NH):
        sl = slice(HD * hh, HD * (hh + 1))
        g = jax.lax.dot_general(qr[:, sl], kr[:, sl],
                                (((1,), (1,)), ((), ())),
                                preferred_element_type=jnp.float32) * scale
        g = jnp.where(allowed, g, -1e30)
        m = jnp.max(g, axis=1, keepdims=True)
        sk = sinks_ref[0:1, hh:hh + 1]
        m2 = jnp.maximum(m, sk)
        pexp = jnp.exp(g - m2)
        denom = jnp.sum(pexp, axis=1, keepdims=True) + jnp.exp(sk - m2)
        probs = pexp / denom
        attn_cols.append(jnp.dot(probs, v[:, sl],
                                 preferred_element_type=jnp.float32))
    attn = jnp.concatenate(attn_cols, axis=1)

    ao = jnp.dot(attn, wo_ref[...],
                 preferred_element_type=jnp.float32) + bo_ref[...]
    hid = x + ao
    v2 = jnp.mean(jnp.square(hid), axis=-1, keepdims=True)
    h2 = hid * jax.lax.rsqrt(v2 + EPS) * ln2_ref[...]

    logits = jnp.dot(h2, rkt_ref[...],
                     preferred_element_type=jnp.float32) + rb_ref[...]
    idx = jax.lax.broadcasted_iota(jnp.int32, (T, E), 1).astype(jnp.float32)
    m1 = jnp.max(logits, axis=1, keepdims=True)
    i1 = jnp.min(jnp.where(logits == m1, idx, 1e9), axis=1, keepdims=True)
    l2 = jnp.where(idx == i1, -1e30, logits)
    m2r = jnp.max(l2, axis=1, keepdims=True)
    i2 = jnp.min(jnp.where(l2 == m2r, idx, 1e9), axis=1, keepdims=True)
    em2 = jnp.exp(m2r - m1)
    p1 = 1.0 / (1.0 + em2)
    p2 = em2 / (1.0 + em2)

    acc = hid
    se_mat = se_s[...]
    for e in range(E):
        if e + NSLOT - 1 < E:
            start_expert(e + NSLOT - 1)
        wait_expert(e)
        g2 = jnp.dot(h2, gup_buf[e % NSLOT],
                     preferred_element_type=jnp.float32) + bgu_ref[e:e + 1, :]
        gl = jnp.minimum(g2, LIMIT)
        gl = gl * jax.nn.sigmoid(gl * ALPHA)
        uc = jnp.clip(g2, -LIMIT, LIMIT) + 1.0
        uc_shift = jnp.concatenate([uc[:, 1:], uc[:, :1]], axis=1)
        fused = jnp.dot(gl * uc_shift, se_mat,
                        preferred_element_type=jnp.float32)
        nxt = jnp.dot(fused, dwn_buf[e % NSLOT],
                      preferred_element_type=jnp.float32) + bd_ref[e:e + 1, :]
        ef = float(e)
        w_e = (jnp.where(i1 == ef, p1, 0.0) + jnp.where(i2 == ef, p2, 0.0))
        acc = acc + nxt * w_e
    out_ref[...] = acc


@jax.jit
def kernel(hidden_states, position_ids, ln1_w, ln2_w, wq, bq, wk, bk, wv, bv,
           wo, bo, sinks, router_kernel, router_bias, gate_up_proj,
           gate_up_proj_bias, down_proj, down_proj_bias):
    x = hidden_states.reshape(T, HID)
    posf = position_ids.astype(jnp.float32).reshape(T, 1)
    jm = jnp.asarray(np.arange(NH * HD) % (HD // 2), jnp.float32)[None, :]
    inv = jnp.exp(jm * (-2.0 / HD) * np.log(THETA))
    ang = posf * inv
    cosf = jnp.cos(ang)
    sinf = jnp.sin(ang)

    vmem = pl.BlockSpec(memory_space=pltpu.VMEM)
    hbm = pl.BlockSpec(memory_space=pl.ANY)

    out = pl.pallas_call(
        _decoder_kernel,
        in_specs=[
            vmem,
            vmem,
            vmem,
            vmem,
            vmem,
            vmem,
            vmem,
            vmem,
            vmem,
            vmem,
            vmem,
            vmem,
            vmem,
            vmem,
            vmem,
            vmem,
            hbm,
            vmem,
            hbm,
            vmem,
        ],
        out_specs=pl.BlockSpec(memory_space=pltpu.VMEM),
        out_shape=jax.ShapeDtypeStruct((T, HID), jnp.float32),
        scratch_shapes=[
            pltpu.VMEM((NSLOT, HID, 2 * FF), jnp.float32),
            pltpu.VMEM((NSLOT, FF, HID), jnp.float32),
            pltpu.VMEM((2 * FF, FF), jnp.float32),
            pltpu.SemaphoreType.DMA((E,)),
            pltpu.SemaphoreType.DMA((E,)),
        ],
    )(x, cosf, sinf, ln1_w[None, :], ln2_w[None, :], wq, bq[None, :], wk,
      bk[None, :], wv, bv[None, :], wo, bo[None, :], sinks[None, :],
      router_kernel.T, router_bias[None, :], gate_up_proj,
      gate_up_proj_bias, down_proj, down_proj_bias)
    return out.reshape(B, S, HID)
